# baseline (speedup 1.0000x reference)
import functools

import jax
import jax.numpy as jnp
from jax import lax
from jax.experimental import pallas as pl
from jax.experimental.pallas import tpu as pltpu
from jax.experimental.pallas import tpu_sc as plsc

_M, _J, _I, _L, _W = 16, 2, 3, 2, 9

_OFF_GAM = 0
_OFF_TMPL = _OFF_GAM + _J * _L * 16
_OFF_HEADW = _OFF_TMPL + _J * _L * 16
_OFF_TAILW = _OFF_HEADW + _J * _I * _L * 16
_OFF_TAILB = _OFF_TAILW + _L * _I * 16
_OFF_S = _OFF_TAILB + _L * 16
_BUF_LEN = _OFF_S + 2 * 16


def _shuffle(vec, idx):
    return lax.gather(
        vec, idx[:, None],
        dimension_numbers=lax.GatherDimensionNumbers(
            offset_dims=(), collapsed_slice_dims=(0,), start_index_map=(0,)),
        slice_sizes=(1,),
        mode=lax.GatherScatterMode.PROMISE_IN_BOUNDS)


def _sc_body(buf_hbm, out_hbm, buf_v, out_v):
    @pl.when((lax.axis_index("c") == 0) & (lax.axis_index("s") == 0))
    def _():
        pltpu.sync_copy(buf_hbm, buf_v)

        def ld(off):
            return buf_v[pl.ds(off, 16)]

        sc0 = ld(_OFF_S)
        sc1 = ld(_OFF_S + 16)
        s = [[_shuffle(sc0 if (w * _L + l) < 16 else sc1,
                       jnp.full((16,), (w * _L + l) % 16, jnp.int32))
              for l in range(_L)] for w in range(_W)]

        captured = [jnp.zeros((16,), jnp.float32) for _ in range(_I)]
        for j in range(_J):
            gam = [ld(_OFF_GAM + (j * _L + l) * 16) for l in range(_L)]
            templ = [ld(_OFF_TMPL + (j * _L + l) * 16) for l in range(_L)]
            sig = [1.0 / (1.0 + jnp.exp(-10.0 * (g - 0.5))) for g in gam]

            best_q = None
            best_w = jnp.zeros((16,), jnp.int32)
            for w in range(_W):
                q = jnp.zeros((16,), jnp.float32)
                for l in range(_L):
                    d = templ[l] - s[w][l]
                    q = q + sig[l] * (d * d)
                if best_q is None:
                    best_q = q
                else:
                    lt = q < best_q
                    best_q = jnp.where(lt, q, best_q)
                    best_w = jnp.where(lt, jnp.full((16,), w, jnp.int32),
                                       best_w)

            zero = jnp.zeros((16,), jnp.float32)
            s_sel = [zero, zero]
            for w in range(_W):
                hit = best_w == jnp.full((16,), w, jnp.int32)
                for l in range(_L):
                    s_sel[l] = s_sel[l] + jnp.where(hit, s[w][l], 0.0)

            for l in range(_L):
                gate = gam[l] > 0.5
                contrib = jnp.where(gate, s_sel[l], 0.0)
                for i in range(_I):
                    hw = ld(_OFF_HEADW + ((j * _I + i) * _L + l) * 16)
                    captured[i] = captured[i] + hw * contrib

        conc = []
        for l in range(_L):
            c = ld(_OFF_TAILB + l * 16)
            for i in range(_I):
                c = c + captured[i] * ld(_OFF_TAILW + (l * _I + i) * 16)
            conc.append(c)

        a = conc[0] * conc[0] + conc[1] * conc[1]
        a = jnp.maximum(a, jnp.float32(1e-20))
        seed_i = lax.shift_right_arithmetic(
            lax.bitcast_convert_type(a, jnp.int32),
            jnp.full((16,), 1, jnp.int32)) + jnp.full((16,), 0x1FBD1DF5,
                                                      jnp.int32)
        y = lax.bitcast_convert_type(seed_i, jnp.float32)
        for _ in range(3):
            y = 0.5 * (y + a / y)

        iota = lax.iota(jnp.int32, 16)
        mask15 = jnp.full((16,), 15, jnp.int32)
        mx = y
        for d in (8, 4, 2, 1):
            ridx = (iota + jnp.full((16,), d, jnp.int32)) & mask15
            mx = jnp.maximum(mx, _shuffle(mx, ridx))
        e = jnp.exp(y - mx)
        tot = e
        for d in (8, 4, 2, 1):
            ridx = (iota + jnp.full((16,), d, jnp.int32)) & mask15
            tot = tot + _shuffle(tot, ridx)
        out_v[...] = e / tot
        pltpu.sync_copy(out_v, out_hbm)


_mesh = plsc.VectorSubcoreMesh(core_axis_name="c", subcore_axis_name="s",
                               num_cores=1)

_sc_call = functools.partial(
    pl.kernel,
    mesh=_mesh,
    out_type=jax.ShapeDtypeStruct((_M,), jnp.float32),
    scratch_types=[
        pltpu.VMEM((_BUF_LEN,), jnp.float32),
        pltpu.VMEM((_M,), jnp.float32),
    ],
)(_sc_body)


def _repack_index():
    import numpy as np
    o_state, o_const, o_gam, o_headw, o_tailw, o_tailb = (
        0, 18, 114, 210, 402, 498)
    idx = np.zeros((_BUF_LEN,), np.int32)
    for j in range(_J):
        for l in range(_L):
            for m in range(_M):
                idx[_OFF_GAM + (j * _L + l) * 16 + m] = (
                    o_gam + m * 6 + (j + 1) * _L + l)
                idx[_OFF_TMPL + (j * _L + l) * 16 + m] = (
                    o_const + m * 6 + j * _L + l)
            for i in range(_I):
                for m in range(_M):
                    idx[_OFF_HEADW + ((j * _I + i) * _L + l) * 16 + m] = (
                        o_headw + m * 12 + (j * _I + i) * _L + l)
    for l in range(_L):
        for i in range(_I):
            for m in range(_M):
                idx[_OFF_TAILW + (l * _I + i) * 16 + m] = (
                    o_tailw + m * 6 + l * _I + i)
        for m in range(_M):
            idx[_OFF_TAILB + l * 16 + m] = o_tailb + m * _L + l
    for k in range(_W * _L):
        idx[_OFF_S + k] = o_state + k
    return idx


_REPACK_IDX = jnp.asarray(_repack_index())


@jax.jit
def kernel(state, constants, gammas, head_W, tail_W, tail_b):
    flat = jnp.concatenate([
        state.reshape(-1), constants.reshape(-1), gammas.reshape(-1),
        head_W.reshape(-1), tail_W.reshape(-1), tail_b.reshape(-1)
    ])
    return _sc_call(flat[_REPACK_IDX])


# speedup vs baseline: 1.2426x; 1.2426x over previous
import functools

import jax
import jax.numpy as jnp
from jax import lax
from jax.experimental import pallas as pl
from jax.experimental.pallas import tpu as pltpu
from jax.experimental.pallas import tpu_sc as plsc

_M, _J, _I, _L, _W = 16, 2, 3, 2, 9

_OFF_GAM = 0
_OFF_TMPL = _OFF_GAM + _J * _L * 16
_OFF_HEADW = _OFF_TMPL + _J * _L * 16
_OFF_TAILW = _OFF_HEADW + _J * _I * _L * 16
_OFF_TAILB = _OFF_TAILW + _L * _I * 16
_OFF_S = _OFF_TAILB + _L * 16
_BUF_LEN = _OFF_S + 2 * 16


def _shuffle(vec, idx):
    return lax.gather(
        vec, idx[:, None],
        dimension_numbers=lax.GatherDimensionNumbers(
            offset_dims=(), collapsed_slice_dims=(0,), start_index_map=(0,)),
        slice_sizes=(1,),
        mode=lax.GatherScatterMode.PROMISE_IN_BOUNDS)


def _sc_body(buf_hbm, out_hbm, buf_v, out_v):
    @pl.when((lax.axis_index("c") == 0) & (lax.axis_index("s") == 0))
    def _():
        pltpu.sync_copy(buf_hbm, buf_v)

        def ld(off):
            return buf_v[pl.ds(off, 16)]

        sc0 = ld(_OFF_S)
        sc1 = ld(_OFF_S + 16)
        s = [[_shuffle(sc0 if (w * _L + l) < 16 else sc1,
                       jnp.full((16,), (w * _L + l) % 16, jnp.int32))
              for l in range(_L)] for w in range(_W)]

        captured = [jnp.zeros((16,), jnp.float32) for _ in range(_I)]
        for j in range(_J):
            gam = [ld(_OFF_GAM + (j * _L + l) * 16) for l in range(_L)]
            templ = [ld(_OFF_TMPL + (j * _L + l) * 16) for l in range(_L)]
            sig = [1.0 / (1.0 + jnp.exp(-10.0 * (g - 0.5))) for g in gam]

            best_q = None
            best_w = jnp.zeros((16,), jnp.int32)
            for w in range(_W):
                q = jnp.zeros((16,), jnp.float32)
                for l in range(_L):
                    d = templ[l] - s[w][l]
                    q = q + sig[l] * (d * d)
                if best_q is None:
                    best_q = q
                else:
                    lt = q < best_q
                    best_q = jnp.where(lt, q, best_q)
                    best_w = jnp.where(lt, jnp.full((16,), w, jnp.int32),
                                       best_w)

            zero = jnp.zeros((16,), jnp.float32)
            s_sel = [zero, zero]
            for w in range(_W):
                hit = best_w == jnp.full((16,), w, jnp.int32)
                for l in range(_L):
                    s_sel[l] = s_sel[l] + jnp.where(hit, s[w][l], 0.0)

            for l in range(_L):
                gate = gam[l] > 0.5
                contrib = jnp.where(gate, s_sel[l], 0.0)
                for i in range(_I):
                    hw = ld(_OFF_HEADW + ((j * _I + i) * _L + l) * 16)
                    captured[i] = captured[i] + hw * contrib

        conc = []
        for l in range(_L):
            c = ld(_OFF_TAILB + l * 16)
            for i in range(_I):
                c = c + captured[i] * ld(_OFF_TAILW + (l * _I + i) * 16)
            conc.append(c)

        a = conc[0] * conc[0] + conc[1] * conc[1]
        a = jnp.maximum(a, jnp.float32(1e-20))
        seed_i = lax.shift_right_arithmetic(
            lax.bitcast_convert_type(a, jnp.int32),
            jnp.full((16,), 1, jnp.int32)) + jnp.full((16,), 0x1FBD1DF5,
                                                      jnp.int32)
        y = lax.bitcast_convert_type(seed_i, jnp.float32)
        for _ in range(3):
            y = 0.5 * (y + a / y)

        iota = lax.iota(jnp.int32, 16)
        mask15 = jnp.full((16,), 15, jnp.int32)
        mx = y
        for d in (8, 4, 2, 1):
            ridx = (iota + jnp.full((16,), d, jnp.int32)) & mask15
            mx = jnp.maximum(mx, _shuffle(mx, ridx))
        e = jnp.exp(y - mx)
        tot = e
        for d in (8, 4, 2, 1):
            ridx = (iota + jnp.full((16,), d, jnp.int32)) & mask15
            tot = tot + _shuffle(tot, ridx)
        out_v[...] = e / tot
        pltpu.sync_copy(out_v, out_hbm)


_mesh = plsc.VectorSubcoreMesh(core_axis_name="c", subcore_axis_name="s",
                               num_cores=1)

_sc_call = functools.partial(
    pl.kernel,
    mesh=_mesh,
    out_type=jax.ShapeDtypeStruct((_M,), jnp.float32),
    scratch_types=[
        pltpu.VMEM((_BUF_LEN,), jnp.float32),
        pltpu.VMEM((_M,), jnp.float32),
    ],
)(_sc_body)


@jax.jit
def kernel(state, constants, gammas, head_W, tail_W, tail_b):
    gam_t = gammas[:, 1:_J + 1, :].transpose(1, 2, 0)
    tmpl_t = constants[:, :_J, :].transpose(1, 2, 0)
    headw_t = head_W.transpose(1, 2, 3, 0)
    tailw_t = tail_W.transpose(1, 2, 0)
    tailb_t = tail_b.transpose(1, 0)
    buf = jnp.concatenate([
        gam_t.reshape(-1), tmpl_t.reshape(-1), headw_t.reshape(-1),
        tailw_t.reshape(-1), tailb_t.reshape(-1), state.reshape(-1),
        jnp.zeros((2 * 16 - _W * _L,), jnp.float32)
    ])
    return _sc_call(buf)
